# baseline (speedup 1.0000x reference)
import functools

import jax
import jax.numpy as jnp
from jax import lax
from jax.experimental import pallas as pl
from jax.experimental.pallas import tpu as pltpu
from jax.experimental.pallas import tpu_sc as plsc

N = 10000
E = 320000
D = 128
D_EMB = 16
H = 8

_HIGH = lax.Precision.HIGHEST


def _linear1_body(x_ref, w_ref, o_ref):
    o_ref[...] = jnp.dot(x_ref[...], w_ref[...], precision=_HIGH)


def _linear1(x, w):
    return pl.pallas_call(
        _linear1_body,
        out_shape=jax.ShapeDtypeStruct((N, D), jnp.float32),
    )(x, w)


_GATHER_WINDOW = 128


def _sc_gather(table, idx):
    idx2 = idx.reshape(1, E)
    mesh = plsc.VectorSubcoreMesh(core_axis_name="core",
                                  subcore_axis_name="subcore")

    @functools.partial(
        pl.kernel,
        out_type=jax.ShapeDtypeStruct((E, D), jnp.float32),
        mesh=mesh,
    )
    def gather_kernel(x_hbm, i_hbm, o_hbm):
        def body(i_vmem, o_vmem):
            pltpu.sync_copy(x_hbm.at[i_vmem.at[0]], o_vmem)

        pltpu.emit_pipeline(
            body,
            grid=(E // _GATHER_WINDOW,),
            in_specs=[pl.BlockSpec((1, _GATHER_WINDOW),
                                   index_map=lambda i: (0, i))],
            out_specs=[pl.BlockSpec((_GATHER_WINDOW, D),
                                    index_map=lambda i: (i, 0))],
            core_axis_name=("core", "subcore"),
            dimension_semantics=(pltpu.PARALLEL,),
        )(i_hbm, o_hbm)

    return gather_kernel(table, idx2)


_EDGE_BLOCK = 2560


def _edge_body(g_ref, emb_ref, a_ref, w0_ref, w1_ref, w2_ref, W2_ref, o_ref):
    h = jax.nn.silu(jnp.dot(emb_ref[...], w0_ref[...], precision=_HIGH))
    h = jax.nn.silu(jnp.dot(h, w1_ref[...], precision=_HIGH))
    t = jnp.dot(h, w2_ref[...], precision=_HIGH)
    m = g_ref[...] * t * a_ref[...]
    o_ref[...] = jax.nn.silu(jnp.dot(m, W2_ref[...], precision=_HIGH))


def _edge_kernel(gathered, emb, attrs, w0, w1, w2, W2):
    nb = E // _EDGE_BLOCK
    return pl.pallas_call(
        _edge_body,
        grid=(nb,),
        in_specs=[
            pl.BlockSpec((_EDGE_BLOCK, D), lambda i: (i, 0)),
            pl.BlockSpec((_EDGE_BLOCK, D_EMB), lambda i: (i, 0)),
            pl.BlockSpec((_EDGE_BLOCK, 1), lambda i: (i, 0)),
            pl.BlockSpec((D_EMB, H), lambda i: (0, 0)),
            pl.BlockSpec((H, H), lambda i: (0, 0)),
            pl.BlockSpec((H, D), lambda i: (0, 0)),
            pl.BlockSpec((D, D), lambda i: (0, 0)),
        ],
        out_specs=pl.BlockSpec((_EDGE_BLOCK, D), lambda i: (i, 0)),
        out_shape=jax.ShapeDtypeStruct((E, D), jnp.float32),
    )(gathered, emb, attrs, w0, w1, w2, W2)


def kernel(node_features, edge_embedding, edge_attrs, edge_index,
           W1, mlp_w0, mlp_w1, mlp_w2, W2):
    edge_src = edge_index[0]
    node_feats = _linear1(node_features, W1)
    gathered = _sc_gather(node_feats, edge_src)
    return _edge_kernel(gathered, edge_embedding, edge_attrs,
                        mlp_w0, mlp_w1, mlp_w2, W2)


# speedup vs baseline: 2.4144x; 2.4144x over previous
import functools

import jax
import jax.numpy as jnp
from jax import lax
from jax.experimental import pallas as pl
from jax.experimental.pallas import tpu as pltpu
from jax.experimental.pallas import tpu_sc as plsc

N = 10000
E = 320000
D = 128
D_EMB = 16
H = 8

_HIGH = lax.Precision.HIGHEST


def _linear1_body(x_ref, w_ref, o_ref):
    o_ref[...] = jnp.dot(x_ref[...], w_ref[...])


def _linear1(x, w):
    return pl.pallas_call(
        _linear1_body,
        out_shape=jax.ShapeDtypeStruct((N, D), jnp.float32),
    )(x, w)


_GATHER_WINDOW = 128


def _sc_gather(table, idx):
    idx2 = idx.reshape(1, E)
    mesh = plsc.VectorSubcoreMesh(core_axis_name="core",
                                  subcore_axis_name="subcore")

    @functools.partial(
        pl.kernel,
        out_type=jax.ShapeDtypeStruct((E, D), jnp.float32),
        mesh=mesh,
    )
    def gather_kernel(x_hbm, i_hbm, o_hbm):
        def body(i_vmem, o_vmem):
            pltpu.sync_copy(x_hbm.at[i_vmem.at[0]], o_vmem)

        pltpu.emit_pipeline(
            body,
            grid=(E // _GATHER_WINDOW,),
            in_specs=[pl.BlockSpec((1, _GATHER_WINDOW),
                                   index_map=lambda i: (0, i))],
            out_specs=[pl.BlockSpec((_GATHER_WINDOW, D),
                                    index_map=lambda i: (i, 0))],
            core_axis_name=("core", "subcore"),
            dimension_semantics=(pltpu.PARALLEL,),
        )(i_hbm, o_hbm)

    return gather_kernel(table, idx2)


_EDGE_BLOCK = 2560


def _edge_body(g_ref, emb_ref, a_ref, w0_ref, w1_ref, w2_ref, W2_ref, o_ref):
    h = jax.nn.silu(jnp.dot(emb_ref[...], w0_ref[...]))
    h = jax.nn.silu(jnp.dot(h, w1_ref[...]))
    t = jnp.dot(h, w2_ref[...])
    m = g_ref[...] * t * a_ref[...]
    o_ref[...] = jax.nn.silu(jnp.dot(m, W2_ref[...]))


def _edge_kernel(gathered, emb, attrs, w0, w1, w2, W2):
    nb = E // _EDGE_BLOCK
    return pl.pallas_call(
        _edge_body,
        grid=(nb,),
        in_specs=[
            pl.BlockSpec((_EDGE_BLOCK, D), lambda i: (i, 0)),
            pl.BlockSpec((_EDGE_BLOCK, D_EMB), lambda i: (i, 0)),
            pl.BlockSpec((_EDGE_BLOCK, 1), lambda i: (i, 0)),
            pl.BlockSpec((D_EMB, H), lambda i: (0, 0)),
            pl.BlockSpec((H, H), lambda i: (0, 0)),
            pl.BlockSpec((H, D), lambda i: (0, 0)),
            pl.BlockSpec((D, D), lambda i: (0, 0)),
        ],
        out_specs=pl.BlockSpec((_EDGE_BLOCK, D), lambda i: (i, 0)),
        out_shape=jax.ShapeDtypeStruct((E, D), jnp.float32),
    )(gathered, emb, attrs, w0, w1, w2, W2)


def kernel(node_features, edge_embedding, edge_attrs, edge_index,
           W1, mlp_w0, mlp_w1, mlp_w2, W2):
    edge_src = edge_index[0]
    node_feats = _linear1(node_features, W1)
    gathered = _sc_gather(node_feats, edge_src)
    return _edge_kernel(gathered, edge_embedding, edge_attrs,
                        mlp_w0, mlp_w1, mlp_w2, W2)


# speedup vs baseline: 2.9614x; 1.2266x over previous
import functools

import jax
import jax.numpy as jnp
from jax import lax
from jax.experimental import pallas as pl
from jax.experimental.pallas import tpu as pltpu
from jax.experimental.pallas import tpu_sc as plsc

N = 10000
E = 320000
D = 128
D_EMB = 16
H = 8

_HIGH = lax.Precision.HIGHEST


def _linear1_body(x_ref, w_ref, o_ref):
    o_ref[...] = jnp.dot(x_ref[...], w_ref[...])


def _linear1(x, w):
    return pl.pallas_call(
        _linear1_body,
        out_shape=jax.ShapeDtypeStruct((N, D), jnp.float32),
    )(x, w)


_GATHER_WINDOW = 128


def _sc_gather(table, idx):
    idx2 = idx.reshape(1, E)
    mesh = plsc.VectorSubcoreMesh(core_axis_name="core",
                                  subcore_axis_name="subcore")

    @functools.partial(
        pl.kernel,
        out_type=jax.ShapeDtypeStruct((E, D), jnp.float32),
        mesh=mesh,
    )
    def gather_kernel(x_hbm, i_hbm, o_hbm):
        def body(i_vmem, o_vmem):
            pltpu.sync_copy(x_hbm.at[i_vmem.at[0]], o_vmem)

        pltpu.emit_pipeline(
            body,
            grid=(E // _GATHER_WINDOW,),
            in_specs=[pl.BlockSpec((1, _GATHER_WINDOW),
                                   index_map=lambda i: (0, i))],
            out_specs=[pl.BlockSpec((_GATHER_WINDOW, D),
                                    index_map=lambda i: (i, 0))],
            core_axis_name=("core", "subcore"),
            dimension_semantics=(pltpu.PARALLEL,),
        )(i_hbm, o_hbm)

    return gather_kernel(table, idx2)


_EDGE_BLOCK = 2560


def _edge_body(g_ref, embT_ref, a_ref, w0T_ref, w1T_ref, w2_ref, W2_ref, o_ref):
    h = jax.nn.silu(jnp.dot(w0T_ref[...], embT_ref[...]))
    h = jax.nn.silu(jnp.dot(w1T_ref[...], h))
    t = lax.dot_general(h, w2_ref[...],
                        (((0,), (0,)), ((), ())))
    m = g_ref[...] * t * a_ref[...]
    o_ref[...] = jax.nn.silu(jnp.dot(m, W2_ref[...]))


def _edge_kernel(gathered, embT, attrs, w0T, w1T, w2, W2):
    nb = E // _EDGE_BLOCK
    return pl.pallas_call(
        _edge_body,
        grid=(nb,),
        in_specs=[
            pl.BlockSpec((_EDGE_BLOCK, D), lambda i: (i, 0)),
            pl.BlockSpec((D_EMB, _EDGE_BLOCK), lambda i: (0, i)),
            pl.BlockSpec((_EDGE_BLOCK, 1), lambda i: (i, 0)),
            pl.BlockSpec((H, D_EMB), lambda i: (0, 0)),
            pl.BlockSpec((H, H), lambda i: (0, 0)),
            pl.BlockSpec((H, D), lambda i: (0, 0)),
            pl.BlockSpec((D, D), lambda i: (0, 0)),
        ],
        out_specs=pl.BlockSpec((_EDGE_BLOCK, D), lambda i: (i, 0)),
        out_shape=jax.ShapeDtypeStruct((E, D), jnp.float32),
    )(gathered, embT, attrs, w0T, w1T, w2, W2)


def kernel(node_features, edge_embedding, edge_attrs, edge_index,
           W1, mlp_w0, mlp_w1, mlp_w2, W2):
    edge_src = edge_index[0]
    embT = edge_embedding.T
    node_feats = _linear1(node_features, W1)
    gathered = _sc_gather(node_feats, edge_src)
    return _edge_kernel(gathered, embT, edge_attrs,
                        mlp_w0.T, mlp_w1.T, mlp_w2, W2)
